# baseline (speedup 1.0000x reference)
import functools
import math

import jax
import jax.numpy as jnp
from jax import lax
from jax.experimental import pallas as pl
from jax.experimental.pallas import tpu as pltpu
from jax.experimental.pallas import tpu_sc as plsc

N_PIX_LO = 128
OVERSAMP_XY = 4
N_PIX_HI = N_PIX_LO * OVERSAMP_XY
NV_LO = 64
OVERSAMP_V = 4
NV_HI = NV_LO * OVERSAMP_V
K_VEL = 8
PIXSCALE_LO = 0.05
PIXSCALE_HI = PIXSCALE_LO / OVERSAMP_XY
DV_LO = 10.0
DV_HI = DV_LO / OVERSAMP_V
VEL0_LO = -0.5 * (NV_LO - 1) * DV_LO
VEL0_HI = VEL0_LO - 0.5 * (DV_LO - DV_HI)
FOV_HALF_HI = 0.5 * (N_PIX_HI - 1) * PIXSCALE_HI
THETA_E = 1.0
R_D = 500.0
V_MAX = 200.0
R_T = 200.0

_NC = 2
_NS = 16
_NW = _NC * _NS
_ROWS_W = N_PIX_HI // _NW
_LROWS_W = _ROWS_W // OVERSAMP_XY
_GROUPS = _ROWS_W * (N_PIX_HI // 16)
_HSIZE = NV_LO * _LROWS_W * N_PIX_LO


def _atan_pos(z):
    t_hi = 2.414213562373095
    t_lo = 0.4142135623730950
    hi = z > t_hi
    mid = z > t_lo
    x = jnp.where(hi, -1.0 / z, jnp.where(mid, (z - 1.0) / (z + 1.0), z))
    w = jnp.where(hi, math.pi / 2.0, jnp.where(mid, math.pi / 4.0, 0.0))
    s = x * x
    p = (((8.05374449538e-2 * s - 1.38776856032e-1) * s + 1.99777106478e-1) * s
         - 3.33329491539e-1) * s * x + x
    return w + p


_FROWS = 64


def _fields_body(params_ref, v_ref, a_ref):
    i = pl.program_id(0)
    f32 = jnp.float32

    cos_i = params_ref[0]
    sin_i = params_ref[1]
    cos_pa = params_ref[2]
    sin_pa = params_ref[3]
    inv_arcsec_per_pc = params_ref[4]
    x0 = params_ref[5]
    y0 = params_ref[6]
    vshift = params_ref[7]
    vlo = params_ref[8]
    vhi = params_ref[9]
    inv_cos_i = params_ref[10]

    col = lax.broadcasted_iota(jnp.int32, (_FROWS, N_PIX_HI), 1).astype(f32)
    row = (lax.broadcasted_iota(jnp.int32, (_FROWS, N_PIX_HI), 0)
           + i * _FROWS).astype(f32)
    thx = -FOV_HALF_HI + PIXSCALE_HI * col
    thy = -FOV_HALF_HI + PIXSCALE_HI * row

    r = jnp.sqrt(thx * thx + thy * thy) + 1e-12
    bx = thx - THETA_E * thx / r
    by = thy - THETA_E * thy / r
    X = (bx - x0) * inv_arcsec_per_pc
    Y = (by - y0) * inv_arcsec_per_pc
    x_gal = cos_pa * X + sin_pa * Y
    y_gal = (-sin_pa * X + cos_pa * Y) * inv_cos_i
    R = jnp.sqrt(x_gal * x_gal + y_gal * y_gal)
    I_map = jnp.exp(-R / R_D)
    v_circ = V_MAX * (2.0 / math.pi) * _atan_pos(R * (1.0 / R_T))
    cos_theta = x_gal / (R + 1e-12)
    v_los = v_circ * sin_i * cos_theta + vshift
    vb = (v_los - VEL0_HI) * (1.0 / DV_HI)
    v_ref[...] = jnp.minimum(jnp.maximum(vb, vlo), vhi)
    a_ref[...] = I_map * (1.0 / (K_VEL * OVERSAMP_V * OVERSAMP_XY * OVERSAMP_XY))


def _fields(params):
    f32 = jnp.float32
    return pl.pallas_call(
        _fields_body,
        grid=(N_PIX_HI // _FROWS,),
        in_specs=[pl.BlockSpec(memory_space=pltpu.SMEM)],
        out_specs=[
            pl.BlockSpec((_FROWS, N_PIX_HI), lambda i: (i, 0)),
            pl.BlockSpec((_FROWS, N_PIX_HI), lambda i: (i, 0)),
        ],
        out_shape=[
            jax.ShapeDtypeStruct((N_PIX_HI, N_PIX_HI), f32),
            jax.ShapeDtypeStruct((N_PIX_HI, N_PIX_HI), f32),
        ],
    )(params)


def _sc_bin_body(v_hbm, a_hbm, dv_hbm, out_hbm, v_v, a_v, dv_v, hist, sem):
    f32 = jnp.float32
    i32 = jnp.int32
    wid = lax.axis_index("s") * _NC + lax.axis_index("c")
    row0 = wid * _ROWS_W

    cp_v = pltpu.async_copy(v_hbm.at[pl.ds(row0, _ROWS_W), :], v_v, sem)
    cp_a = pltpu.async_copy(a_hbm.at[pl.ds(row0, _ROWS_W), :], a_v, sem)
    pltpu.sync_copy(dv_hbm, dv_v)

    z16 = jnp.zeros((16,), f32)

    @plsc.parallel_loop(0, _HSIZE // 16, 1, unroll=4)
    def zero(i):
        hist[pl.ds(pl.multiple_of(i << 4, 16), 16)] = z16

    cp_v.wait()
    cp_a.wait()

    lane = lax.broadcasted_iota(i32, (16,), 0)
    aks = [dv_v[k, :] for k in range(K_VEL)]

    @plsc.parallel_loop(0, _GROUPS, 1, unroll=2)
    def group(g):
        hi_row = g >> 5
        x0 = pl.multiple_of((g & 31) << 4, 16)
        base = (((hi_row >> 2) << 7) + ((x0 + lane) >> 2)).astype(i32)
        v = v_v[hi_row, pl.ds(x0, 16)]
        a = a_v[hi_row, pl.ds(x0, 16)]
        for k in range(K_VEL):
            c = v + aks[k]
            iv0 = c.astype(i32)
            fv = c - iv0.astype(f32)
            idx0 = ((iv0 & ~3) << 7) + base
            idx1 = (((iv0 + 1) & ~3) << 7) + base
            w1 = a * fv
            plsc.addupdate_scatter(hist, [idx0], a - w1)
            plsc.addupdate_scatter(hist, [idx1], w1)

    pltpu.sync_copy(hist, out_hbm.at[pl.ds(wid * _HSIZE, _HSIZE)])


@functools.partial(
    pl.kernel,
    mesh=plsc.VectorSubcoreMesh(core_axis_name="c", subcore_axis_name="s"),
    compiler_params=pltpu.CompilerParams(needs_layout_passes=False),
    out_type=jax.ShapeDtypeStruct((_NW * _HSIZE,), jnp.float32),
    scratch_types=[
        pltpu.VMEM((_ROWS_W, N_PIX_HI), jnp.float32),
        pltpu.VMEM((_ROWS_W, N_PIX_HI), jnp.float32),
        pltpu.VMEM((K_VEL, 16), jnp.float32),
        pltpu.VMEM((_HSIZE,), jnp.float32),
        pltpu.SemaphoreType.DMA,
    ],
)
def _sc_bin(v_hbm, a_hbm, dv_hbm, out_hbm, v_v, a_v, dv_v, hist, sem):
    _sc_bin_body(v_hbm, a_hbm, dv_hbm, out_hbm, v_v, a_v, dv_v, hist, sem)


def kernel(inclination, sky_rot, line_broadening, velocity_shift, x0, y0, distance_pc):
    f32 = jnp.float32
    cos_i = jnp.cos(inclination)
    sin_i = jnp.sin(inclination)
    pa = sky_rot + math.pi / 2.0
    cos_pa = jnp.cos(pa)
    sin_pa = jnp.sin(pa)
    arcsec_per_pc = 206265.0 / distance_pc
    inv_arcsec_per_pc = distance_pc * (1.0 / 206265.0)
    inv_cos_i = 1.0 / (cos_i + 1e-12)

    sigma = jnp.abs(line_broadening) + 1e-12
    p_mid = (jnp.arange(K_VEL, dtype=f32) + 0.5) / K_VEL
    unit = math.sqrt(2.0) * jax.scipy.special.erfinv(2.0 * p_mid - 1.0)
    dv_off = sigma * unit

    amax = jnp.max(jnp.abs(dv_off)) * (1.0 / DV_HI)
    vlo = amax
    vhi = (NV_HI - 1) - 1e-3 - amax
    params = jnp.concatenate([
        jnp.stack([cos_i, sin_i, cos_pa, sin_pa, inv_arcsec_per_pc,
                   x0, y0, velocity_shift, vlo, vhi, inv_cos_i]).astype(f32),
        dv_off.astype(f32),
    ])

    v_los, amp = _fields(params)
    dv16 = jnp.broadcast_to((dv_off * (1.0 / DV_HI)).astype(f32).reshape(K_VEL, 1),
                            (K_VEL, 16))
    flat = _sc_bin(v_los, amp, dv16)
    return (flat.reshape(_NW, NV_LO, _LROWS_W, N_PIX_LO)
            .transpose(1, 0, 2, 3)
            .reshape(NV_LO, N_PIX_LO, N_PIX_LO))


# speedup vs baseline: 1.1162x; 1.1162x over previous
import functools
import math

import jax
import jax.numpy as jnp
from jax import lax
from jax.experimental import pallas as pl
from jax.experimental.pallas import tpu as pltpu
from jax.experimental.pallas import tpu_sc as plsc

N_PIX_LO = 128
OVERSAMP_XY = 4
N_PIX_HI = N_PIX_LO * OVERSAMP_XY
NV_LO = 64
OVERSAMP_V = 4
NV_HI = NV_LO * OVERSAMP_V
K_VEL = 8
PIXSCALE_LO = 0.05
PIXSCALE_HI = PIXSCALE_LO / OVERSAMP_XY
DV_LO = 10.0
DV_HI = DV_LO / OVERSAMP_V
VEL0_LO = -0.5 * (NV_LO - 1) * DV_LO
VEL0_HI = VEL0_LO - 0.5 * (DV_LO - DV_HI)
FOV_HALF_HI = 0.5 * (N_PIX_HI - 1) * PIXSCALE_HI
THETA_E = 1.0
R_D = 500.0
V_MAX = 200.0
R_T = 200.0

_NC = 2
_NS = 16
_NW = _NC * _NS
_ROWS_W = N_PIX_HI // _NW
_LROWS_W = _ROWS_W // OVERSAMP_XY
_GROUPS = _ROWS_W * (N_PIX_HI // 16)
_HSIZE = NV_LO * _LROWS_W * N_PIX_LO


def _atan_pos(z):
    t_hi = 2.414213562373095
    t_lo = 0.4142135623730950
    hi = z > t_hi
    mid = z > t_lo
    x = jnp.where(hi, -1.0 / z, jnp.where(mid, (z - 1.0) / (z + 1.0), z))
    w = jnp.where(hi, math.pi / 2.0, jnp.where(mid, math.pi / 4.0, 0.0))
    s = x * x
    p = (((8.05374449538e-2 * s - 1.38776856032e-1) * s + 1.99777106478e-1) * s
         - 3.33329491539e-1) * s * x + x
    return w + p


_FROWS = 64


def _fields_body(params_ref, v_ref, a_ref):
    i = pl.program_id(0)
    f32 = jnp.float32

    cos_i = params_ref[0]
    sin_i = params_ref[1]
    cos_pa = params_ref[2]
    sin_pa = params_ref[3]
    inv_arcsec_per_pc = params_ref[4]
    x0 = params_ref[5]
    y0 = params_ref[6]
    vshift = params_ref[7]
    vlo = params_ref[8]
    vhi = params_ref[9]
    inv_cos_i = params_ref[10]

    col = lax.broadcasted_iota(jnp.int32, (_FROWS, N_PIX_HI), 1).astype(f32)
    row = (lax.broadcasted_iota(jnp.int32, (_FROWS, N_PIX_HI), 0)
           + i * _FROWS).astype(f32)
    thx = -FOV_HALF_HI + PIXSCALE_HI * col
    thy = -FOV_HALF_HI + PIXSCALE_HI * row

    r = jnp.sqrt(thx * thx + thy * thy) + 1e-12
    bx = thx - THETA_E * thx / r
    by = thy - THETA_E * thy / r
    X = (bx - x0) * inv_arcsec_per_pc
    Y = (by - y0) * inv_arcsec_per_pc
    x_gal = cos_pa * X + sin_pa * Y
    y_gal = (-sin_pa * X + cos_pa * Y) * inv_cos_i
    R = jnp.sqrt(x_gal * x_gal + y_gal * y_gal)
    I_map = jnp.exp(-R / R_D)
    v_circ = V_MAX * (2.0 / math.pi) * _atan_pos(R * (1.0 / R_T))
    cos_theta = x_gal / (R + 1e-12)
    v_los = v_circ * sin_i * cos_theta + vshift
    vb = (v_los - VEL0_HI) * (1.0 / DV_HI)
    v_ref[...] = jnp.minimum(jnp.maximum(vb, vlo), vhi)
    a_ref[...] = I_map * (1.0 / (K_VEL * OVERSAMP_V * OVERSAMP_XY * OVERSAMP_XY))


def _fields(params):
    f32 = jnp.float32
    return pl.pallas_call(
        _fields_body,
        grid=(N_PIX_HI // _FROWS,),
        in_specs=[pl.BlockSpec(memory_space=pltpu.SMEM)],
        out_specs=[
            pl.BlockSpec((_FROWS, N_PIX_HI), lambda i: (i, 0)),
            pl.BlockSpec((_FROWS, N_PIX_HI), lambda i: (i, 0)),
        ],
        out_shape=[
            jax.ShapeDtypeStruct((N_PIX_HI, N_PIX_HI), f32),
            jax.ShapeDtypeStruct((N_PIX_HI, N_PIX_HI), f32),
        ],
    )(params)


def _sc_bin_body(v_hbm, a_hbm, dv_hbm, out_hbm, v_v, a_v, dv_v, hist):
    f32 = jnp.float32
    i32 = jnp.int32
    wid = lax.axis_index("s") * _NC + lax.axis_index("c")
    row0 = wid * _ROWS_W

    pltpu.sync_copy(v_hbm.at[pl.ds(row0, _ROWS_W), :], v_v)
    pltpu.sync_copy(a_hbm.at[pl.ds(row0, _ROWS_W), :], a_v)
    pltpu.sync_copy(dv_hbm, dv_v)

    z16 = jnp.zeros((16,), f32)

    @plsc.parallel_loop(0, _HSIZE // 16, 1, unroll=4)
    def zero(i):
        hist[pl.ds(pl.multiple_of(i << 4, 16), 16)] = z16

    lane = lax.broadcasted_iota(i32, (16,), 0)
    aks = [dv_v[k, :] for k in range(K_VEL)]

    @plsc.parallel_loop(0, _GROUPS, 1, unroll=2)
    def group(g):
        hi_row = g >> 5
        x0 = pl.multiple_of((g & 31) << 4, 16)
        base = (((hi_row >> 2) << 7) + ((x0 + lane) >> 2)).astype(i32)
        v = v_v[hi_row, pl.ds(x0, 16)]
        a = a_v[hi_row, pl.ds(x0, 16)]
        for k in range(K_VEL):
            c = v + aks[k]
            iv0 = c.astype(i32)
            fv = c - iv0.astype(f32)
            idx0 = ((iv0 & ~3) << 7) + base
            idx1 = (((iv0 + 1) & ~3) << 7) + base
            w1 = a * fv
            plsc.addupdate_scatter(hist, [idx0], a - w1)
            plsc.addupdate_scatter(hist, [idx1], w1)

    pltpu.sync_copy(hist, out_hbm.at[pl.ds(wid * _HSIZE, _HSIZE)])


@functools.partial(
    pl.kernel,
    mesh=plsc.VectorSubcoreMesh(core_axis_name="c", subcore_axis_name="s"),
    compiler_params=pltpu.CompilerParams(needs_layout_passes=False),
    out_type=jax.ShapeDtypeStruct((_NW * _HSIZE,), jnp.float32),
    scratch_types=[
        pltpu.VMEM((_ROWS_W, N_PIX_HI), jnp.float32),
        pltpu.VMEM((_ROWS_W, N_PIX_HI), jnp.float32),
        pltpu.VMEM((K_VEL, 16), jnp.float32),
        pltpu.VMEM((_HSIZE,), jnp.float32),
    ],
)
def _sc_bin(v_hbm, a_hbm, dv_hbm, out_hbm, v_v, a_v, dv_v, hist):
    _sc_bin_body(v_hbm, a_hbm, dv_hbm, out_hbm, v_v, a_v, dv_v, hist)


def kernel(inclination, sky_rot, line_broadening, velocity_shift, x0, y0, distance_pc):
    f32 = jnp.float32
    cos_i = jnp.cos(inclination)
    sin_i = jnp.sin(inclination)
    pa = sky_rot + math.pi / 2.0
    cos_pa = jnp.cos(pa)
    sin_pa = jnp.sin(pa)
    arcsec_per_pc = 206265.0 / distance_pc
    inv_arcsec_per_pc = distance_pc * (1.0 / 206265.0)
    inv_cos_i = 1.0 / (cos_i + 1e-12)

    sigma = jnp.abs(line_broadening) + 1e-12
    p_mid = (jnp.arange(K_VEL, dtype=f32) + 0.5) / K_VEL
    unit = math.sqrt(2.0) * jax.scipy.special.erfinv(2.0 * p_mid - 1.0)
    dv_off = sigma * unit

    amax = jnp.max(jnp.abs(dv_off)) * (1.0 / DV_HI)
    vlo = amax
    vhi = (NV_HI - 1) - 1e-3 - amax
    params = jnp.concatenate([
        jnp.stack([cos_i, sin_i, cos_pa, sin_pa, inv_arcsec_per_pc,
                   x0, y0, velocity_shift, vlo, vhi, inv_cos_i]).astype(f32),
        dv_off.astype(f32),
    ])

    v_los, amp = _fields(params)
    dv16 = jnp.broadcast_to((dv_off * (1.0 / DV_HI)).astype(f32).reshape(K_VEL, 1),
                            (K_VEL, 16))
    flat = _sc_bin(v_los, amp, dv16)
    return (flat.reshape(_NW, NV_LO, _LROWS_W, N_PIX_LO)
            .transpose(1, 0, 2, 3)
            .reshape(NV_LO, N_PIX_LO, N_PIX_LO))


# speedup vs baseline: 1.1414x; 1.0225x over previous
import functools
import math

import jax
import jax.numpy as jnp
from jax import lax
from jax.experimental import pallas as pl
from jax.experimental.pallas import tpu as pltpu
from jax.experimental.pallas import tpu_sc as plsc

N_PIX_LO = 128
OVERSAMP_XY = 4
N_PIX_HI = N_PIX_LO * OVERSAMP_XY
NV_LO = 64
OVERSAMP_V = 4
NV_HI = NV_LO * OVERSAMP_V
K_VEL = 8
PIXSCALE_LO = 0.05
PIXSCALE_HI = PIXSCALE_LO / OVERSAMP_XY
DV_LO = 10.0
DV_HI = DV_LO / OVERSAMP_V
VEL0_LO = -0.5 * (NV_LO - 1) * DV_LO
VEL0_HI = VEL0_LO - 0.5 * (DV_LO - DV_HI)
FOV_HALF_HI = 0.5 * (N_PIX_HI - 1) * PIXSCALE_HI
THETA_E = 1.0
R_D = 500.0
V_MAX = 200.0
R_T = 200.0

_NC = 2
_NS = 16
_NW = _NC * _NS
_ROWS_W = N_PIX_HI // _NW
_LROWS_W = _ROWS_W // OVERSAMP_XY
_GROUPS = _ROWS_W * (N_PIX_HI // 16)
_HSIZE = NV_LO * _LROWS_W * N_PIX_LO


def _atan_pos(z):
    t_hi = 2.414213562373095
    t_lo = 0.4142135623730950
    hi = z > t_hi
    mid = z > t_lo
    x = jnp.where(hi, -1.0 / z, jnp.where(mid, (z - 1.0) / (z + 1.0), z))
    w = jnp.where(hi, math.pi / 2.0, jnp.where(mid, math.pi / 4.0, 0.0))
    s = x * x
    p = (((8.05374449538e-2 * s - 1.38776856032e-1) * s + 1.99777106478e-1) * s
         - 3.33329491539e-1) * s * x + x
    return w + p


_FROWS = 64


def _fields_body(params_ref, v_ref, a_ref):
    i = pl.program_id(0)
    f32 = jnp.float32

    cos_i = params_ref[0]
    sin_i = params_ref[1]
    cos_pa = params_ref[2]
    sin_pa = params_ref[3]
    inv_arcsec_per_pc = params_ref[4]
    x0 = params_ref[5]
    y0 = params_ref[6]
    vshift = params_ref[7]
    vlo = params_ref[8]
    vhi = params_ref[9]
    inv_cos_i = params_ref[10]

    m = lax.broadcasted_iota(jnp.int32, (_FROWS, N_PIX_HI), 1)
    col = (((m & 127) << 2) + (m >> 7)).astype(f32)
    row = (lax.broadcasted_iota(jnp.int32, (_FROWS, N_PIX_HI), 0)
           + i * _FROWS).astype(f32)
    thx = -FOV_HALF_HI + PIXSCALE_HI * col
    thy = -FOV_HALF_HI + PIXSCALE_HI * row

    r = jnp.sqrt(thx * thx + thy * thy) + 1e-12
    bx = thx - THETA_E * thx / r
    by = thy - THETA_E * thy / r
    X = (bx - x0) * inv_arcsec_per_pc
    Y = (by - y0) * inv_arcsec_per_pc
    x_gal = cos_pa * X + sin_pa * Y
    y_gal = (-sin_pa * X + cos_pa * Y) * inv_cos_i
    R = jnp.sqrt(x_gal * x_gal + y_gal * y_gal)
    I_map = jnp.exp(-R / R_D)
    v_circ = V_MAX * (2.0 / math.pi) * _atan_pos(R * (1.0 / R_T))
    cos_theta = x_gal / (R + 1e-12)
    v_los = v_circ * sin_i * cos_theta + vshift
    vb = (v_los - VEL0_HI) * (1.0 / DV_HI)
    v_ref[...] = jnp.minimum(jnp.maximum(vb, vlo), vhi)
    a_ref[...] = I_map * (1.0 / (K_VEL * OVERSAMP_V * OVERSAMP_XY * OVERSAMP_XY))


def _fields(params):
    f32 = jnp.float32
    return pl.pallas_call(
        _fields_body,
        grid=(N_PIX_HI // _FROWS,),
        in_specs=[pl.BlockSpec(memory_space=pltpu.SMEM)],
        out_specs=[
            pl.BlockSpec((_FROWS, N_PIX_HI), lambda i: (i, 0)),
            pl.BlockSpec((_FROWS, N_PIX_HI), lambda i: (i, 0)),
        ],
        out_shape=[
            jax.ShapeDtypeStruct((N_PIX_HI, N_PIX_HI), f32),
            jax.ShapeDtypeStruct((N_PIX_HI, N_PIX_HI), f32),
        ],
    )(params)


def _sc_bin_body(v_hbm, a_hbm, dv_hbm, out_hbm, v_v, a_v, dv_v, hist, sem):
    f32 = jnp.float32
    i32 = jnp.int32
    wid = lax.axis_index("s") * _NC + lax.axis_index("c")
    row0 = wid * _ROWS_W

    cp_v = pltpu.async_copy(v_hbm.at[pl.ds(row0, _ROWS_W), :], v_v, sem)
    cp_a = pltpu.async_copy(a_hbm.at[pl.ds(row0, _ROWS_W), :], a_v, sem)
    pltpu.sync_copy(dv_hbm, dv_v)

    z16 = jnp.zeros((16,), f32)

    @plsc.parallel_loop(0, _HSIZE // 16, 1, unroll=4)
    def zero(i):
        hist[pl.ds(pl.multiple_of(i << 4, 16), 16)] = z16

    cp_v.wait()
    cp_a.wait()

    lane = lax.broadcasted_iota(i32, (16,), 0)
    aks = [dv_v[k, :] for k in range(K_VEL)]

    @plsc.parallel_loop(0, _GROUPS, 1, unroll=2)
    def group(g):
        hi_row = g >> 5
        x0 = pl.multiple_of((g & 31) << 4, 16)
        base = ((hi_row >> 2) << 7) + ((g & 7) << 4) + lane
        v = v_v[hi_row, pl.ds(x0, 16)]
        a = a_v[hi_row, pl.ds(x0, 16)]
        for k in range(K_VEL):
            c = v + aks[k]
            iv0 = c.astype(i32)
            fv = c - iv0.astype(f32)
            idx0 = ((iv0 & ~3) << 7) + base
            idx1 = (((iv0 + 1) & ~3) << 7) + base
            w1 = a * fv
            plsc.addupdate_scatter(hist, [idx0], a - w1)
            plsc.addupdate_scatter(hist, [idx1], w1)

    pltpu.sync_copy(hist, out_hbm.at[pl.ds(wid * _HSIZE, _HSIZE)])


@functools.partial(
    pl.kernel,
    mesh=plsc.VectorSubcoreMesh(core_axis_name="c", subcore_axis_name="s"),
    compiler_params=pltpu.CompilerParams(needs_layout_passes=False),
    out_type=jax.ShapeDtypeStruct((_NW * _HSIZE,), jnp.float32),
    scratch_types=[
        pltpu.VMEM((_ROWS_W, N_PIX_HI), jnp.float32),
        pltpu.VMEM((_ROWS_W, N_PIX_HI), jnp.float32),
        pltpu.VMEM((K_VEL, 16), jnp.float32),
        pltpu.VMEM((_HSIZE,), jnp.float32),
        pltpu.SemaphoreType.DMA,
    ],
)
def _sc_bin(v_hbm, a_hbm, dv_hbm, out_hbm, v_v, a_v, dv_v, hist, sem):
    _sc_bin_body(v_hbm, a_hbm, dv_hbm, out_hbm, v_v, a_v, dv_v, hist, sem)


def kernel(inclination, sky_rot, line_broadening, velocity_shift, x0, y0, distance_pc):
    f32 = jnp.float32
    cos_i = jnp.cos(inclination)
    sin_i = jnp.sin(inclination)
    pa = sky_rot + math.pi / 2.0
    cos_pa = jnp.cos(pa)
    sin_pa = jnp.sin(pa)
    inv_arcsec_per_pc = distance_pc * (1.0 / 206265.0)
    inv_cos_i = 1.0 / (cos_i + 1e-12)

    sigma = jnp.abs(line_broadening) + 1e-12
    p_mid = (jnp.arange(K_VEL, dtype=f32) + 0.5) / K_VEL
    unit = math.sqrt(2.0) * jax.scipy.special.erfinv(2.0 * p_mid - 1.0)
    dv_off = sigma * unit

    amax = jnp.max(jnp.abs(dv_off)) * (1.0 / DV_HI)
    vlo = amax
    vhi = (NV_HI - 1) - 1e-3 - amax
    params = jnp.concatenate([
        jnp.stack([cos_i, sin_i, cos_pa, sin_pa, inv_arcsec_per_pc,
                   x0, y0, velocity_shift, vlo, vhi, inv_cos_i]).astype(f32),
        dv_off.astype(f32),
    ])

    v_los, amp = _fields(params)
    dv16 = jnp.broadcast_to((dv_off * (1.0 / DV_HI)).astype(f32).reshape(K_VEL, 1),
                            (K_VEL, 16))
    flat = _sc_bin(v_los, amp, dv16)
    return (flat.reshape(_NW, NV_LO, _LROWS_W, N_PIX_LO)
            .transpose(1, 0, 2, 3)
            .reshape(NV_LO, N_PIX_LO, N_PIX_LO))


# speedup vs baseline: 1.1681x; 1.0234x over previous
import functools
import math

import jax
import jax.numpy as jnp
from jax import lax
from jax.experimental import pallas as pl
from jax.experimental.pallas import tpu as pltpu
from jax.experimental.pallas import tpu_sc as plsc

N_PIX_LO = 128
OVERSAMP_XY = 4
N_PIX_HI = N_PIX_LO * OVERSAMP_XY
NV_LO = 64
OVERSAMP_V = 4
NV_HI = NV_LO * OVERSAMP_V
K_VEL = 8
PIXSCALE_LO = 0.05
PIXSCALE_HI = PIXSCALE_LO / OVERSAMP_XY
DV_LO = 10.0
DV_HI = DV_LO / OVERSAMP_V
VEL0_LO = -0.5 * (NV_LO - 1) * DV_LO
VEL0_HI = VEL0_LO - 0.5 * (DV_LO - DV_HI)
FOV_HALF_HI = 0.5 * (N_PIX_HI - 1) * PIXSCALE_HI
THETA_E = 1.0
R_D = 500.0
V_MAX = 200.0
R_T = 200.0

_NC = 2
_NS = 16
_NW = _NC * _NS
_ROWS_W = N_PIX_HI // _NW
_LROWS_W = _ROWS_W // OVERSAMP_XY
_GROUPS = _ROWS_W * (N_PIX_HI // 16)
_HSIZE = NV_LO * _LROWS_W * N_PIX_LO


def _atan_pos(z):
    t_hi = 2.414213562373095
    t_lo = 0.4142135623730950
    hi = z > t_hi
    mid = z > t_lo
    x = jnp.where(hi, -1.0 / z, jnp.where(mid, (z - 1.0) / (z + 1.0), z))
    w = jnp.where(hi, math.pi / 2.0, jnp.where(mid, math.pi / 4.0, 0.0))
    s = x * x
    p = (((8.05374449538e-2 * s - 1.38776856032e-1) * s + 1.99777106478e-1) * s
         - 3.33329491539e-1) * s * x + x
    return w + p


_FROWS = 256


def _fields_body(params_ref, v_ref, a_ref):
    i = pl.program_id(0)
    f32 = jnp.float32

    cos_i = params_ref[0]
    sin_i = params_ref[1]
    cos_pa = params_ref[2]
    sin_pa = params_ref[3]
    inv_arcsec_per_pc = params_ref[4]
    x0 = params_ref[5]
    y0 = params_ref[6]
    vshift = params_ref[7]
    vlo = params_ref[8]
    vhi = params_ref[9]
    inv_cos_i = params_ref[10]

    m = lax.broadcasted_iota(jnp.int32, (_FROWS, N_PIX_HI), 1)
    col = (((m & 127) << 2) + (m >> 7)).astype(f32)
    row = (lax.broadcasted_iota(jnp.int32, (_FROWS, N_PIX_HI), 0)
           + i * _FROWS).astype(f32)
    thx = -FOV_HALF_HI + PIXSCALE_HI * col
    thy = -FOV_HALF_HI + PIXSCALE_HI * row

    r = jnp.sqrt(thx * thx + thy * thy) + 1e-12
    bx = thx - THETA_E * thx / r
    by = thy - THETA_E * thy / r
    X = (bx - x0) * inv_arcsec_per_pc
    Y = (by - y0) * inv_arcsec_per_pc
    x_gal = cos_pa * X + sin_pa * Y
    y_gal = (-sin_pa * X + cos_pa * Y) * inv_cos_i
    R = jnp.sqrt(x_gal * x_gal + y_gal * y_gal)
    I_map = jnp.exp(-R / R_D)
    v_circ = V_MAX * (2.0 / math.pi) * _atan_pos(R * (1.0 / R_T))
    cos_theta = x_gal / (R + 1e-12)
    v_los = v_circ * sin_i * cos_theta + vshift
    vb = (v_los - VEL0_HI) * (1.0 / DV_HI)
    v_ref[...] = jnp.minimum(jnp.maximum(vb, vlo), vhi)
    a_ref[...] = I_map * (1.0 / (K_VEL * OVERSAMP_V * OVERSAMP_XY * OVERSAMP_XY))


def _fields(params):
    f32 = jnp.float32
    return pl.pallas_call(
        _fields_body,
        grid=(N_PIX_HI // _FROWS,),
        in_specs=[pl.BlockSpec(memory_space=pltpu.SMEM)],
        out_specs=[
            pl.BlockSpec((_FROWS, N_PIX_HI), lambda i: (i, 0)),
            pl.BlockSpec((_FROWS, N_PIX_HI), lambda i: (i, 0)),
        ],
        out_shape=[
            jax.ShapeDtypeStruct((N_PIX_HI, N_PIX_HI), f32),
            jax.ShapeDtypeStruct((N_PIX_HI, N_PIX_HI), f32),
        ],
    )(params)


def _sc_bin_body(v_hbm, a_hbm, dv_hbm, out_hbm, v_v, a_v, dv_v, hist, sem):
    f32 = jnp.float32
    i32 = jnp.int32
    wid = lax.axis_index("s") * _NC + lax.axis_index("c")
    row0 = wid * _ROWS_W

    cp_v = pltpu.async_copy(v_hbm.at[pl.ds(row0, _ROWS_W), :], v_v, sem)
    cp_a = pltpu.async_copy(a_hbm.at[pl.ds(row0, _ROWS_W), :], a_v, sem)
    pltpu.sync_copy(dv_hbm, dv_v)

    z16 = jnp.zeros((16,), f32)

    @plsc.parallel_loop(0, _HSIZE // 16, 1, unroll=4)
    def zero(i):
        hist[pl.ds(pl.multiple_of(i << 4, 16), 16)] = z16

    cp_v.wait()
    cp_a.wait()

    lane = lax.broadcasted_iota(i32, (16,), 0)
    aks = [dv_v[k, :] for k in range(K_VEL)]

    @plsc.parallel_loop(0, _GROUPS, 1, unroll=2)
    def group(g):
        hi_row = g >> 5
        x0 = pl.multiple_of((g & 31) << 4, 16)
        base = ((hi_row >> 2) << 7) + ((g & 7) << 4) + lane
        v = v_v[hi_row, pl.ds(x0, 16)]
        a = a_v[hi_row, pl.ds(x0, 16)]
        for k in range(K_VEL):
            c = v + aks[k]
            iv0 = c.astype(i32)
            fv = c - iv0.astype(f32)
            idx0 = ((iv0 & ~3) << 7) + base
            idx1 = (((iv0 + 1) & ~3) << 7) + base
            w1 = a * fv
            plsc.addupdate_scatter(hist, [idx0], a - w1)
            plsc.addupdate_scatter(hist, [idx1], w1)

    pltpu.sync_copy(hist, out_hbm.at[pl.ds(wid * _HSIZE, _HSIZE)])


@functools.partial(
    pl.kernel,
    mesh=plsc.VectorSubcoreMesh(core_axis_name="c", subcore_axis_name="s"),
    compiler_params=pltpu.CompilerParams(needs_layout_passes=False),
    out_type=jax.ShapeDtypeStruct((_NW * _HSIZE,), jnp.float32),
    scratch_types=[
        pltpu.VMEM((_ROWS_W, N_PIX_HI), jnp.float32),
        pltpu.VMEM((_ROWS_W, N_PIX_HI), jnp.float32),
        pltpu.VMEM((K_VEL, 16), jnp.float32),
        pltpu.VMEM((_HSIZE,), jnp.float32),
        pltpu.SemaphoreType.DMA,
    ],
)
def _sc_bin(v_hbm, a_hbm, dv_hbm, out_hbm, v_v, a_v, dv_v, hist, sem):
    _sc_bin_body(v_hbm, a_hbm, dv_hbm, out_hbm, v_v, a_v, dv_v, hist, sem)


def kernel(inclination, sky_rot, line_broadening, velocity_shift, x0, y0, distance_pc):
    f32 = jnp.float32
    cos_i = jnp.cos(inclination)
    sin_i = jnp.sin(inclination)
    pa = sky_rot + math.pi / 2.0
    cos_pa = jnp.cos(pa)
    sin_pa = jnp.sin(pa)
    inv_arcsec_per_pc = distance_pc * (1.0 / 206265.0)
    inv_cos_i = 1.0 / (cos_i + 1e-12)

    sigma = jnp.abs(line_broadening) + 1e-12
    p_mid = (jnp.arange(K_VEL, dtype=f32) + 0.5) / K_VEL
    unit = math.sqrt(2.0) * jax.scipy.special.erfinv(2.0 * p_mid - 1.0)
    dv_off = sigma * unit

    amax = jnp.max(jnp.abs(dv_off)) * (1.0 / DV_HI)
    vlo = amax
    vhi = (NV_HI - 1) - 1e-3 - amax
    params = jnp.concatenate([
        jnp.stack([cos_i, sin_i, cos_pa, sin_pa, inv_arcsec_per_pc,
                   x0, y0, velocity_shift, vlo, vhi, inv_cos_i]).astype(f32),
        dv_off.astype(f32),
    ])

    v_los, amp = _fields(params)
    dv16 = jnp.broadcast_to((dv_off * (1.0 / DV_HI)).astype(f32).reshape(K_VEL, 1),
                            (K_VEL, 16))
    flat = _sc_bin(v_los, amp, dv16)
    return (flat.reshape(_NW, NV_LO, _LROWS_W, N_PIX_LO)
            .transpose(1, 0, 2, 3)
            .reshape(NV_LO, N_PIX_LO, N_PIX_LO))
